# Initial kernel scaffold; baseline (speedup 1.0000x reference)
import jax
import jax.numpy as jnp
from jax.experimental import pallas as pl


def kernel(hidden_states, W):
    raise NotImplementedError("write your pallas kernel here")


# speedup vs baseline: 1.1434x; 1.1434x over previous
import jax
import jax.numpy as jnp
from jax.experimental import pallas as pl

TOP_K = 8
NUM_EXPERTS = 64
BLOCK_T = 1024


def _router_body(x_ref, w_ref, logits_ref, weights_ref, experts_ref):
    x = x_ref[...]
    w = w_ref[...]
    logits = jax.lax.dot_general(
        x, w, (((1,), (1,)), ((), ())), preferred_element_type=jnp.float32
    )
    logits_ref[...] = logits
    bt = logits.shape[0]
    iota = jax.lax.broadcasted_iota(jnp.int32, (bt, NUM_EXPERTS), 1)
    work = logits
    vals, idxs = [], []
    for _ in range(TOP_K):
        m = jnp.max(work, axis=-1, keepdims=True)
        hit = work == m
        idx = jnp.min(jnp.where(hit, iota, NUM_EXPERTS), axis=-1, keepdims=True)
        vals.append(m)
        idxs.append(idx)
        work = jnp.where(iota == idx, -jnp.inf, work)
    topv = jnp.concatenate(vals, axis=-1)
    topi = jnp.concatenate(idxs, axis=-1)
    e = jnp.exp(topv - topv[:, :1])
    weights_ref[...] = e / jnp.sum(e, axis=-1, keepdims=True)
    experts_ref[...] = topi


def kernel(hidden_states, W):
    nt, hd = hidden_states.shape
    ne = W.shape[0]
    grid = (nt // BLOCK_T,)
    logits, weights, experts = pl.pallas_call(
        _router_body,
        grid=grid,
        in_specs=[
            pl.BlockSpec((BLOCK_T, hd), lambda i: (i, 0)),
            pl.BlockSpec((ne, hd), lambda i: (0, 0)),
        ],
        out_specs=[
            pl.BlockSpec((BLOCK_T, ne), lambda i: (i, 0)),
            pl.BlockSpec((BLOCK_T, TOP_K), lambda i: (i, 0)),
            pl.BlockSpec((BLOCK_T, TOP_K), lambda i: (i, 0)),
        ],
        out_shape=[
            jax.ShapeDtypeStruct((nt, ne), jnp.float32),
            jax.ShapeDtypeStruct((nt, TOP_K), jnp.float32),
            jax.ShapeDtypeStruct((nt, TOP_K), jnp.int32),
        ],
    )(hidden_states, W)
    return (weights, experts, logits)
